# baseline (speedup 1.0000x reference)
import functools

import jax
import jax.numpy as jnp
from jax import lax
from jax.experimental import pallas as pl
from jax.experimental.pallas import tpu as pltpu
from jax.experimental.pallas import tpu_sc as plsc

K = 512
B2 = 2 * K
LSTRIDE = B2 + 8
HWORDS = 16 * LSTRIDE
NC, NS = 2, 16
NW = NC * NS
N = 400000
PER_W = 12800
CHUNK = 1600
NVEC = CHUNK // 16
SCALE = 511.99

_mesh = plsc.VectorSubcoreMesh(
    core_axis_name="c", subcore_axis_name="s", num_cores=NC, num_subcores=NS
)


@functools.partial(
    pl.kernel,
    out_type=jax.ShapeDtypeStruct((NW, B2), jnp.int32),
    mesh=_mesh,
    scratch_types=[
        pltpu.VMEM((CHUNK,), jnp.float32),
        pltpu.VMEM((CHUNK,), jnp.float32),
        pltpu.VMEM((CHUNK,), jnp.int32),
        pltpu.VMEM((CHUNK,), jnp.int32),
        pltpu.VMEM((HWORDS,), jnp.int32),
        pltpu.VMEM((B2,), jnp.int32),
        pltpu.SemaphoreType.DMA,
        pltpu.SemaphoreType.DMA,
        pltpu.SemaphoreType.DMA,
        pltpu.SemaphoreType.DMA,
    ],
    compiler_params=pltpu.CompilerParams(needs_layout_passes=False),
)
def _hist(p_hbm, lab_hbm, t_hbm, pb0, pb1, lb0, lb1, hist, tloc,
          psem0, psem1, lsem0, lsem1):
    c = lax.axis_index("c")
    s = lax.axis_index("s")
    w = c * NS + s
    iot = lax.iota(jnp.int32, 16)
    lane_base = iot * LSTRIDE
    dump = lane_base + B2 + 2
    ones = jnp.ones((16,), jnp.int32)
    zeros = jnp.zeros((16,), jnp.int32)

    base = w * PER_W
    npair = jnp.where(w == NW - 1, (N - (NW - 1) * PER_W) // (2 * CHUNK),
                      PER_W // (2 * CHUNK))

    def _copies(ci, pb, lb, psem, lsem):
        off = base + ci * CHUNK
        return (
            pltpu.make_async_copy(p_hbm.at[pl.ds(off, CHUNK)], pb, psem),
            pltpu.make_async_copy(lab_hbm.at[pl.ds(off, CHUNK)], lb, lsem),
        )

    def _start(ci, pb, lb, psem, lsem):
        cp, cl = _copies(ci, pb, lb, psem, lsem)
        cp.start()
        cl.start()

    def _wait(ci, pb, lb, psem, lsem):
        cp, cl = _copies(ci, pb, lb, psem, lsem)
        cp.wait()
        cl.wait()

    _start(0, pb0, lb0, psem0, lsem0)

    def _zero(i, carry):
        for u in range(8):
            hist[pl.ds(i * 128 + u * 16, 16)] = zeros
        return carry

    lax.fori_loop(0, HWORDS // 128, _zero, 0)

    def _bucket(p, lb):
        q = jnp.where(lb == 2, 2.0 - p, p)
        bi = (q * jnp.float32(SCALE)).astype(jnp.int32)
        ix = bi + lane_base
        return jnp.where(lb == 0, dump, ix)

    def _consume(pb, lb):
        def _vec(v, carry2):
            ps = [pb[pl.ds(v * 64 + 16 * u, 16)] for u in range(4)]
            ls = [lb[pl.ds(v * 64 + 16 * u, 16)] for u in range(4)]
            ixs = [_bucket(p, l) for p, l in zip(ps, ls)]
            for ix in ixs:
                plsc.addupdate_scatter(hist, [ix], ones)
            return carry2

        lax.fori_loop(0, NVEC // 4, _vec, 0)

    def _pair(i, carry):
        _start(2 * i + 1, pb1, lb1, psem1, lsem1)
        _wait(2 * i, pb0, lb0, psem0, lsem0)
        _consume(pb0, lb0)

        @pl.when(i + 1 < npair)
        def _():
            _start(2 * i + 2, pb0, lb0, psem0, lsem0)

        _wait(2 * i + 1, pb1, lb1, psem1, lsem1)
        _consume(pb1, lb1)
        return carry

    lax.fori_loop(0, npair, _pair, 0)

    def _fold(g, carry):
        acc = hist[pl.ds(g * 16, 16)]
        for r in range(1, 16):
            acc = acc + hist[pl.ds(r * LSTRIDE + g * 16, 16)]
        tloc[pl.ds(g * 16, 16)] = acc
        return carry

    lax.fori_loop(0, B2 // 16, _fold, 0)

    pltpu.sync_copy(tloc, t_hbm.at[w])


def _scan_body(t_ref, o_ref):
    h = jnp.sum(t_ref[...].astype(jnp.float32), axis=0, keepdims=True)
    h0 = h[:, :K]
    h1 = h[:, K:]
    bi = lax.broadcasted_iota(jnp.int32, (K, K), 0)
    bj = lax.broadcasted_iota(jnp.int32, (K, K), 1)
    suf = (bi >= bj).astype(jnp.float32)
    f0 = jnp.dot(h0, suf, preferred_element_type=jnp.float32)
    f1 = jnp.dot(h1, suf, preferred_element_type=jnp.float32)
    g = jnp.sum(h1)
    den = g + f0
    j = 1.0 - (g - f1) / jnp.maximum(den, 1.0)
    j = jnp.where(den == 0.0, 0.0, j)
    col = lax.broadcasted_iota(jnp.int32, (1, K), 1)
    jlast = jnp.sum(jnp.where(col == 0, j, 0.0))
    o_ref[0, 0] = (jnp.sum(j) - 0.5 * jlast) * jnp.float32(1.0 / K)


_scan_tc = pl.pallas_call(
    _scan_body,
    out_shape=jax.ShapeDtypeStruct((1, 1), jnp.float32),
    out_specs=pl.BlockSpec(memory_space=pltpu.SMEM),
)


def kernel(probas, labels):
    p = probas[:, 2, :].reshape(-1)
    lab = labels.reshape(-1).astype(jnp.int32)
    t = _hist(p, lab)
    out = _scan_tc(t)
    return out[0, 0]
